# Initial kernel scaffold; baseline (speedup 1.0000x reference)
import jax
import jax.numpy as jnp
from jax.experimental import pallas as pl


def kernel(feats, proto):
    raise NotImplementedError("write your pallas kernel here")


# speedup vs baseline: 9.6070x; 9.6070x over previous
import jax
import jax.numpy as jnp
from jax.experimental import pallas as pl

_NUM_CLASSES = 1024
_NUM_PROTOS = 3
_FEAT_DIM = 256
_TAU = 0.1
_BN = 512


def _proto_logits_kernel(f_ref, p_ref, out_ref):
    f = f_ref[...]
    fn = jnp.sqrt(jnp.sum(f * f, axis=1, keepdims=True))
    fs = f / jnp.maximum(fn, 1e-8)
    acc = None
    for k in range(_NUM_PROTOS):
        pk = p_ref[k]
        pn = jnp.sqrt(jnp.sum(pk * pk, axis=1, keepdims=True))
        ps = pk / jnp.maximum(pn, 1e-8)
        lk = jax.lax.dot_general(
            fs, ps, (((1,), (1,)), ((), ())), preferred_element_type=jnp.float32
        )
        acc = lk if acc is None else jnp.maximum(acc, lk)
    out_ref[...] = acc * (1.0 / _TAU)


def kernel(feats, proto):
    n = feats.shape[0]
    p = proto.transpose(1, 0, 2)
    return pl.pallas_call(
        _proto_logits_kernel,
        grid=(n // _BN,),
        in_specs=[
            pl.BlockSpec((_BN, _FEAT_DIM), lambda i: (i, 0)),
            pl.BlockSpec((_NUM_PROTOS, _NUM_CLASSES, _FEAT_DIM), lambda i: (0, 0, 0)),
        ],
        out_specs=pl.BlockSpec((_BN, _NUM_CLASSES), lambda i: (i, 0)),
        out_shape=jax.ShapeDtypeStruct((n, _NUM_CLASSES), jnp.float32),
    )(feats, p)


# speedup vs baseline: 10.2059x; 1.0623x over previous
import jax
import jax.numpy as jnp
from jax.experimental import pallas as pl
from jax.experimental.pallas import tpu as pltpu

_NUM_CLASSES = 1024
_NUM_PROTOS = 3
_FEAT_DIM = 256
_TAU = 0.1
_BN = 512


def _proto_logits_kernel(f_ref, p_ref, out_ref, ps_ref):
    @pl.when(pl.program_id(0) == 0)
    def _():
        p = p_ref[...]
        pn = jnp.sqrt(jnp.sum(p * p, axis=2, keepdims=True))
        ps_ref[...] = p / jnp.maximum(pn, 1e-8)

    f = f_ref[...]
    fn = jnp.sqrt(jnp.sum(f * f, axis=1, keepdims=True))
    fs = f / jnp.maximum(fn, 1e-8)
    acc = None
    for k in range(_NUM_PROTOS):
        lk = jax.lax.dot_general(
            fs, ps_ref[k], (((1,), (1,)), ((), ())), preferred_element_type=jnp.float32
        )
        acc = lk if acc is None else jnp.maximum(acc, lk)
    out_ref[...] = acc * (1.0 / _TAU)


def kernel(feats, proto):
    n = feats.shape[0]
    p = proto.transpose(1, 0, 2)
    return pl.pallas_call(
        _proto_logits_kernel,
        grid=(n // _BN,),
        in_specs=[
            pl.BlockSpec((_BN, _FEAT_DIM), lambda i: (i, 0)),
            pl.BlockSpec((_NUM_PROTOS, _NUM_CLASSES, _FEAT_DIM), lambda i: (0, 0, 0)),
        ],
        out_specs=pl.BlockSpec((_BN, _NUM_CLASSES), lambda i: (i, 0)),
        out_shape=jax.ShapeDtypeStruct((n, _NUM_CLASSES), jnp.float32),
        scratch_shapes=[
            pltpu.VMEM((_NUM_PROTOS, _NUM_CLASSES, _FEAT_DIM), jnp.float32)
        ],
    )(feats, p)


# speedup vs baseline: 12.5342x; 1.2281x over previous
import jax
import jax.numpy as jnp
from jax.experimental import pallas as pl
from jax.experimental.pallas import tpu as pltpu

_NUM_CLASSES = 1024
_NUM_PROTOS = 3
_FEAT_DIM = 256
_TAU = 0.1
_BN = 1024


def _proto_logits_kernel(f_ref, p_ref, out_ref, ps_ref):
    @pl.when(pl.program_id(0) == 0)
    def _():
        p = p_ref[...]
        pn = jnp.sqrt(jnp.sum(p * p, axis=2, keepdims=True))
        ps_ref[...] = p / jnp.maximum(pn, 1e-8)

    f = f_ref[...]
    fn = jnp.sqrt(jnp.sum(f * f, axis=1, keepdims=True))
    fs = f / jnp.maximum(fn, 1e-8)
    acc = None
    for k in range(_NUM_PROTOS):
        lk = jax.lax.dot_general(
            fs, ps_ref[k], (((1,), (1,)), ((), ())), preferred_element_type=jnp.float32
        )
        acc = lk if acc is None else jnp.maximum(acc, lk)
    out_ref[...] = acc * (1.0 / _TAU)


def kernel(feats, proto):
    n = feats.shape[0]
    p = proto.transpose(1, 0, 2)
    return pl.pallas_call(
        _proto_logits_kernel,
        grid=(n // _BN,),
        in_specs=[
            pl.BlockSpec((_BN, _FEAT_DIM), lambda i: (i, 0)),
            pl.BlockSpec((_NUM_PROTOS, _NUM_CLASSES, _FEAT_DIM), lambda i: (0, 0, 0)),
        ],
        out_specs=pl.BlockSpec((_BN, _NUM_CLASSES), lambda i: (i, 0)),
        out_shape=jax.ShapeDtypeStruct((n, _NUM_CLASSES), jnp.float32),
        scratch_shapes=[
            pltpu.VMEM((_NUM_PROTOS, _NUM_CLASSES, _FEAT_DIM), jnp.float32)
        ],
    )(feats, p)


# speedup vs baseline: 13.5314x; 1.0796x over previous
import jax
import jax.numpy as jnp
from jax.experimental import pallas as pl
from jax.experimental.pallas import tpu as pltpu

_NUM_CLASSES = 1024
_NUM_PROTOS = 3
_FEAT_DIM = 256
_TAU = 0.1
_BN = 2048


def _proto_logits_kernel(f_ref, p_ref, out_ref, ps_ref):
    @pl.when(pl.program_id(0) == 0)
    def _():
        p = p_ref[...]
        pn = jnp.sqrt(jnp.sum(p * p, axis=2, keepdims=True))
        ps_ref[...] = p / jnp.maximum(pn, 1e-8)

    f = f_ref[...]
    fn = jnp.sqrt(jnp.sum(f * f, axis=1, keepdims=True))
    fs = f / jnp.maximum(fn, 1e-8)
    acc = None
    for k in range(_NUM_PROTOS):
        lk = jax.lax.dot_general(
            fs, ps_ref[k], (((1,), (1,)), ((), ())), preferred_element_type=jnp.float32
        )
        acc = lk if acc is None else jnp.maximum(acc, lk)
    out_ref[...] = acc * (1.0 / _TAU)


def kernel(feats, proto):
    n = feats.shape[0]
    p = proto.transpose(1, 0, 2)
    return pl.pallas_call(
        _proto_logits_kernel,
        grid=(n // _BN,),
        in_specs=[
            pl.BlockSpec((_BN, _FEAT_DIM), lambda i: (i, 0)),
            pl.BlockSpec((_NUM_PROTOS, _NUM_CLASSES, _FEAT_DIM), lambda i: (0, 0, 0)),
        ],
        out_specs=pl.BlockSpec((_BN, _NUM_CLASSES), lambda i: (i, 0)),
        out_shape=jax.ShapeDtypeStruct((n, _NUM_CLASSES), jnp.float32),
        scratch_shapes=[
            pltpu.VMEM((_NUM_PROTOS, _NUM_CLASSES, _FEAT_DIM), jnp.float32)
        ],
    )(feats, p)
